# baseline (speedup 1.0000x reference)
import functools

import jax
import jax.numpy as jnp
from jax import lax
from jax.experimental import pallas as pl
from jax.experimental.pallas import tpu as pltpu
from jax.experimental.pallas import tpu_sc as plsc

VOCAB = 1024
D_MODEL = 1024
SEQ_LEN = 2048
BATCH = 4
MAX_TOK = 20
SCALE = 8.0 ** 0.5

NC = 2
NS = 16
NW = NC * NS
LANES = 16

W_PER_B = NW // BATCH
D_PER_W = D_MODEL // W_PER_B
D_CHUNK = 16
N_CHUNK = D_PER_W // D_CHUNK

TBL_ROWS = 24
TBL = TBL_ROWS * D_MODEL
OUT_CHUNK = D_CHUNK * SEQ_LEN


def _sc_body(x_hbm, w_hbm, out_hbm, tbl_v, idx_v, out_v):
    wid = lax.axis_index("s") * NC + lax.axis_index("c")
    b = wid // W_PER_B
    d0 = (wid % W_PER_B) * D_PER_W

    pltpu.sync_copy(w_hbm.at[pl.ds(0, TBL)], tbl_v)

    def scale_row(r, _):
        def scale_vec(j, _):
            p = r * D_MODEL + j * LANES
            tbl_v[pl.ds(p, LANES)] = tbl_v[pl.ds(p, LANES)] * SCALE
            return 0
        lax.fori_loop(0, D_MODEL // LANES, scale_vec, 0)
        return 0
    lax.fori_loop(0, MAX_TOK, scale_row, 0)

    def zero_vec(j, _):
        tbl_v[pl.ds(MAX_TOK * D_MODEL + j * LANES, LANES)] = jnp.zeros(
            (LANES,), jnp.float32)
        return 0
    lax.fori_loop(0, D_MODEL // LANES, zero_vec, 0)

    pltpu.sync_copy(x_hbm.at[pl.ds(b * SEQ_LEN, SEQ_LEN)], idx_v)

    def clamp_vec(g, _):
        v = idx_v[pl.ds(g * LANES, LANES)]
        v = jnp.where(v < MAX_TOK, v, MAX_TOK) * D_MODEL
        idx_v[pl.ds(g * LANES, LANES)] = v
        return 0
    lax.fori_loop(0, SEQ_LEN // LANES, clamp_vec, 0)

    for c in range(N_CHUNK):
        dbase = d0 + c * D_CHUNK

        def gather_grp(g, _, dbase=dbase):
            idxs = idx_v[pl.ds(g * LANES, LANES)]
            for dd in range(D_CHUNK):
                rows = plsc.load_gather(
                    tbl_v, [idxs + jnp.full((LANES,), dbase + dd, jnp.int32)])
                out_v[pl.ds(dd * SEQ_LEN + g * LANES, LANES)] = rows
            return 0
        lax.fori_loop(0, SEQ_LEN // LANES, gather_grp, 0)

        base = b * (D_MODEL * SEQ_LEN) + dbase * SEQ_LEN
        pltpu.sync_copy(out_v, out_hbm.at[pl.ds(base, OUT_CHUNK)])


_sc_embed = functools.partial(
    pl.kernel,
    mesh=plsc.VectorSubcoreMesh(
        core_axis_name="c", subcore_axis_name="s",
        num_cores=NC, num_subcores=NS),
    out_type=jax.ShapeDtypeStruct((BATCH * D_MODEL * SEQ_LEN,), jnp.float32),
    compiler_params=pltpu.CompilerParams(needs_layout_passes=False),
    scratch_types=[
        pltpu.VMEM((TBL,), jnp.float32),
        pltpu.VMEM((SEQ_LEN,), jnp.int32),
        pltpu.VMEM((OUT_CHUNK,), jnp.float32),
    ],
)(_sc_body)


def kernel(x, embed_w):
    xt = jnp.transpose(x).astype(jnp.int32).reshape(-1)
    w_flat = embed_w.reshape(-1)
    out = _sc_embed(xt, w_flat)
    return out.reshape(BATCH, D_MODEL, SEQ_LEN)


# speedup vs baseline: 1.6049x; 1.6049x over previous
import functools

import jax
import jax.numpy as jnp
from jax import lax
from jax.experimental import pallas as pl
from jax.experimental.pallas import tpu as pltpu
from jax.experimental.pallas import tpu_sc as plsc

VOCAB = 1024
D_MODEL = 1024
SEQ_LEN = 2048
BATCH = 4
MAX_TOK = 20
SCALE = 8.0 ** 0.5

NC = 2
NS = 16
NW = NC * NS
LANES = 16

W_PER_B = NW // BATCH
D_PER_W = D_MODEL // W_PER_B
D_CHUNK = 16
N_CHUNK = D_PER_W // D_CHUNK
N_GRP = SEQ_LEN // LANES

TBL_ROWS = 24
TBL = TBL_ROWS * D_MODEL
OUT_CHUNK = D_CHUNK * SEQ_LEN


def _sc_body(x_hbm, w_hbm, out_hbm, tbl_v, idx_v, out0_v, out1_v,
             sem0, sem1):
    wid = lax.axis_index("s") * NC + lax.axis_index("c")
    b = wid // W_PER_B
    d0 = (wid % W_PER_B) * D_PER_W

    pltpu.sync_copy(w_hbm.at[pl.ds(0, TBL)], tbl_v)

    @plsc.parallel_loop(0, MAX_TOK * (D_MODEL // LANES))
    def _scale(i):
        p = i * LANES
        tbl_v[pl.ds(p, LANES)] = tbl_v[pl.ds(p, LANES)] * SCALE

    @plsc.parallel_loop(0, D_MODEL // LANES)
    def _zero(j):
        tbl_v[pl.ds(MAX_TOK * D_MODEL + j * LANES, LANES)] = jnp.zeros(
            (LANES,), jnp.float32)

    pltpu.sync_copy(x_hbm.at[pl.ds(b * SEQ_LEN, SEQ_LEN)], idx_v)

    @plsc.parallel_loop(0, N_GRP)
    def _clamp(g):
        v = idx_v[pl.ds(g * LANES, LANES)]
        idx_v[pl.ds(g * LANES, LANES)] = jnp.where(v < MAX_TOK, v, MAX_TOK) * D_MODEL

    bufs = (out0_v, out1_v)
    sems = (sem0, sem1)
    copies = [None, None]
    for c in range(N_CHUNK):
        dbase = d0 + c * D_CHUNK
        nbuf = c % 2
        if copies[nbuf] is not None:
            copies[nbuf].wait()

        out_v = bufs[nbuf]

        @plsc.parallel_loop(0, N_GRP, unroll=2)
        def _gather(g, dbase=dbase, out_v=out_v):
            idxs = idx_v[pl.ds(g * LANES, LANES)]
            for dd in range(D_CHUNK):
                rows = plsc.load_gather(
                    tbl_v, [idxs + jnp.full((LANES,), dbase + dd, jnp.int32)])
                out_v[pl.ds(dd * SEQ_LEN + g * LANES, LANES)] = rows

        base = b * (D_MODEL * SEQ_LEN) + dbase * SEQ_LEN
        copies[nbuf] = pltpu.async_copy(
            out_v, out_hbm.at[pl.ds(base, OUT_CHUNK)], sems[nbuf])

    copies[0].wait()
    copies[1].wait()


_sc_embed = functools.partial(
    pl.kernel,
    mesh=plsc.VectorSubcoreMesh(
        core_axis_name="c", subcore_axis_name="s",
        num_cores=NC, num_subcores=NS),
    out_type=jax.ShapeDtypeStruct((BATCH * D_MODEL * SEQ_LEN,), jnp.float32),
    compiler_params=pltpu.CompilerParams(needs_layout_passes=False),
    scratch_types=[
        pltpu.VMEM((TBL,), jnp.float32),
        pltpu.VMEM((SEQ_LEN,), jnp.int32),
        pltpu.VMEM((OUT_CHUNK,), jnp.float32),
        pltpu.VMEM((OUT_CHUNK,), jnp.float32),
        pltpu.SemaphoreType.DMA,
        pltpu.SemaphoreType.DMA,
    ],
)(_sc_body)


def kernel(x, embed_w):
    xt = jnp.transpose(x).astype(jnp.int32).reshape(-1)
    w_flat = embed_w.reshape(-1)
    out = _sc_embed(xt, w_flat)
    return out.reshape(BATCH, D_MODEL, SEQ_LEN)


# speedup vs baseline: 1.9852x; 1.2370x over previous
import functools

import jax
import jax.numpy as jnp
from jax import lax
from jax.experimental import pallas as pl
from jax.experimental.pallas import tpu as pltpu
from jax.experimental.pallas import tpu_sc as plsc

VOCAB = 1024
D_MODEL = 1024
SEQ_LEN = 2048
BATCH = 4
MAX_TOK = 20
SCALE = 8.0 ** 0.5

NC = 2
NS = 16
NW = NC * NS
LANES = 16

W_PER_B = NW // BATCH
D_PER_W = D_MODEL // W_PER_B
D_CHUNK = 16
N_CHUNK = D_PER_W // D_CHUNK
N_GRP = SEQ_LEN // LANES

TBL_ROWS = 24
TBL = TBL_ROWS * D_MODEL


def _sc_body(x_hbm, w_hbm, out_hbm, x_v, tbl_v, idx_v, out0_v, out1_v,
             sem0, sem1):
    wid = lax.axis_index("s") * NC + lax.axis_index("c")
    b = wid // W_PER_B
    d0 = (wid % W_PER_B) * D_PER_W

    for r in range(TBL_ROWS):
        pltpu.sync_copy(w_hbm.at[r], tbl_v.at[pl.ds(r * D_MODEL, D_MODEL)])
    pltpu.sync_copy(x_hbm, x_v)

    @plsc.parallel_loop(0, MAX_TOK * (D_MODEL // LANES), unroll=4)
    def _scale(i):
        p = i * LANES
        tbl_v[pl.ds(p, LANES)] = tbl_v[pl.ds(p, LANES)] * SCALE

    @plsc.parallel_loop(0, D_MODEL // LANES, unroll=4)
    def _zero(j):
        tbl_v[pl.ds(MAX_TOK * D_MODEL + j * LANES, LANES)] = jnp.zeros(
            (LANES,), jnp.float32)

    lane_step = lax.iota(jnp.int32, LANES) * BATCH

    @plsc.parallel_loop(0, N_GRP, unroll=2)
    def _extract(g):
        flat = g * (LANES * BATCH) + b + lane_step
        v = plsc.load_gather(x_v, [flat])
        idx_v[pl.ds(g * LANES, LANES)] = jnp.minimum(v, MAX_TOK) * D_MODEL

    bufs = (out0_v, out1_v)
    sems = (sem0, sem1)
    copies = [None, None]
    for c in range(N_CHUNK):
        dbase = d0 + c * D_CHUNK
        nbuf = c % 2
        if copies[nbuf] is not None:
            copies[nbuf].wait()

        out_v = bufs[nbuf]

        @plsc.parallel_loop(0, N_GRP, unroll=4)
        def _gather(g, dbase=dbase, out_v=out_v):
            rows = idx_v[pl.ds(g * LANES, LANES)]
            for dd in range(D_CHUNK):
                out_v[dd, pl.ds(g * LANES, LANES)] = plsc.load_gather(
                    tbl_v, [rows + jnp.full((LANES,), dbase + dd, jnp.int32)])

        copies[nbuf] = pltpu.async_copy(
            out_v, out_hbm.at[b, pl.ds(dbase, D_CHUNK), :], sems[nbuf])

    copies[0].wait()
    copies[1].wait()


_sc_embed = functools.partial(
    pl.kernel,
    mesh=plsc.VectorSubcoreMesh(
        core_axis_name="c", subcore_axis_name="s",
        num_cores=NC, num_subcores=NS),
    out_type=jax.ShapeDtypeStruct((BATCH, D_MODEL, SEQ_LEN), jnp.float32),
    compiler_params=pltpu.CompilerParams(needs_layout_passes=False),
    scratch_types=[
        pltpu.VMEM((SEQ_LEN * BATCH,), jnp.int32),
        pltpu.VMEM((TBL,), jnp.float32),
        pltpu.VMEM((SEQ_LEN,), jnp.int32),
        pltpu.VMEM((D_CHUNK, SEQ_LEN), jnp.float32),
        pltpu.VMEM((D_CHUNK, SEQ_LEN), jnp.float32),
        pltpu.SemaphoreType.DMA,
        pltpu.SemaphoreType.DMA,
    ],
)(_sc_body)


def kernel(x, embed_w):
    return _sc_embed(x.astype(jnp.int32).reshape(-1), embed_w)
